# Initial kernel scaffold; baseline (speedup 1.0000x reference)
import jax
import jax.numpy as jnp
from jax.experimental import pallas as pl


def kernel(decoder_states, attn_copy, src_token_ids, w_copy, b_copy, w_gen, b_gen):
    raise NotImplementedError("write your pallas kernel here")


# speedup vs baseline: 4.8666x; 4.8666x over previous
import functools

import jax
import jax.numpy as jnp
from jax.experimental import pallas as pl
from jax.experimental.pallas import tpu as pltpu


def _pick_vt(v: int) -> int:
    best = 128
    for d in range(128, 1281, 128):
        if v % d == 0:
            best = d
    return best


def _stats_kernel(x_ref, wg_ref, bg_ref, wc_ref, bc_ref,
                  m_out, s_out, g_out, m_sc, s_sc):
    k = pl.program_id(1)
    nk = pl.num_programs(1)
    x = x_ref[...]
    logits = jnp.dot(x.astype(jnp.bfloat16), wg_ref[...].astype(jnp.bfloat16),
                     preferred_element_type=jnp.float32) + bg_ref[...]

    @pl.when(k == 0)
    def _():
        m_sc[...] = jnp.full_like(m_sc, -1e30)
        s_sc[...] = jnp.zeros_like(s_sc)
        gate_logit = jnp.sum(x * wc_ref[...], axis=-1, keepdims=True) + bc_ref[0, 0]
        g_out[...] = jax.nn.sigmoid(gate_logit)

    m_old = m_sc[...]
    m_new = jnp.maximum(m_old, jnp.max(logits, axis=-1, keepdims=True))
    s_sc[...] = (s_sc[...] * jnp.exp(m_old - m_new)
                 + jnp.sum(jnp.exp(logits - m_new), axis=-1, keepdims=True))
    m_sc[...] = m_new

    @pl.when(k == nk - 1)
    def _():
        m_out[...] = m_sc[...]
        s_out[...] = s_sc[...]


def _blend_kernel(x_ref, wg_ref, bg_ref, attn_ref, ids_ref,
                  m_ref, s_ref, g_ref, o_ref, *, vt: int):
    k = pl.program_id(1)
    s_len = attn_ref.shape[1]
    logits = jnp.dot(x_ref[...].astype(jnp.bfloat16),
                     wg_ref[...].astype(jnp.bfloat16),
                     preferred_element_type=jnp.float32) + bg_ref[...]
    g = g_ref[...]
    probs_scaled = jnp.exp(logits - m_ref[...]) * (g / s_ref[...])
    iota = jax.lax.broadcasted_iota(jnp.int32, (s_len, vt), 1) + k * vt
    onehot = jnp.where(ids_ref[0] == iota, 1.0, 0.0)
    copy_tile = jnp.dot(attn_ref[...], onehot,
                        preferred_element_type=jnp.float32)
    o_ref[...] = probs_scaled + (1.0 - g) * copy_tile


def kernel(decoder_states, attn_copy, src_token_ids, w_copy, b_copy, w_gen, b_gen):
    n, l, d = decoder_states.shape
    s = attn_copy.shape[-1]
    v = w_gen.shape[-1]
    vt = _pick_vt(v)
    kt = v // vt
    rows = n * l

    x = decoder_states.reshape(rows, d)
    attn = attn_copy.reshape(rows, s)
    ids = src_token_ids.astype(jnp.int32).reshape(n, s, 1)
    wc_row = w_copy.reshape(1, d)
    bc = b_copy.reshape(1, 1)
    bg = b_gen.reshape(1, v)

    col = jax.ShapeDtypeStruct((rows, 1), jnp.float32)
    m, se, g = pl.pallas_call(
        _stats_kernel,
        grid=(n, kt),
        in_specs=[
            pl.BlockSpec((l, d), lambda i, k: (i, 0)),
            pl.BlockSpec((d, vt), lambda i, k: (0, k)),
            pl.BlockSpec((1, vt), lambda i, k: (0, k)),
            pl.BlockSpec((1, d), lambda i, k: (0, 0)),
            pl.BlockSpec((1, 1), lambda i, k: (0, 0)),
        ],
        out_specs=[
            pl.BlockSpec((l, 1), lambda i, k: (i, 0)),
            pl.BlockSpec((l, 1), lambda i, k: (i, 0)),
            pl.BlockSpec((l, 1), lambda i, k: (i, 0)),
        ],
        out_shape=[col, col, col],
        scratch_shapes=[
            pltpu.VMEM((l, 1), jnp.float32),
            pltpu.VMEM((l, 1), jnp.float32),
        ],
        compiler_params=pltpu.CompilerParams(
            dimension_semantics=("parallel", "arbitrary"),
            vmem_limit_bytes=50 * 1024 * 1024,
        ),
    )(x, w_gen, bg, wc_row, bc)

    out = pl.pallas_call(
        functools.partial(_blend_kernel, vt=vt),
        grid=(n, kt),
        in_specs=[
            pl.BlockSpec((l, d), lambda i, k: (i, 0)),
            pl.BlockSpec((d, vt), lambda i, k: (0, k)),
            pl.BlockSpec((1, vt), lambda i, k: (0, k)),
            pl.BlockSpec((l, s), lambda i, k: (i, 0)),
            pl.BlockSpec((1, s, 1), lambda i, k: (i, 0, 0)),
            pl.BlockSpec((l, 1), lambda i, k: (i, 0)),
            pl.BlockSpec((l, 1), lambda i, k: (i, 0)),
            pl.BlockSpec((l, 1), lambda i, k: (i, 0)),
        ],
        out_specs=pl.BlockSpec((l, vt), lambda i, k: (i, k)),
        out_shape=jax.ShapeDtypeStruct((rows, v), jnp.float32),
        compiler_params=pltpu.CompilerParams(
            dimension_semantics=("parallel", "arbitrary"),
            vmem_limit_bytes=50 * 1024 * 1024,
        ),
    )(x, w_gen, bg, attn, ids, m, se, g)

    return out.reshape(n, l, v)
